# initial kernel scaffold (unmeasured)
import jax
import jax.numpy as jnp
from jax import lax
from jax.experimental import pallas as pl
from jax.experimental.pallas import tpu as pltpu


def kernel(
    x,
):
    def body(*refs):
        pass

    out_shape = jax.ShapeDtypeStruct(..., jnp.float32)
    return pl.pallas_call(body, out_shape=out_shape)(...)


# baseline (device time: 74562 ns/iter reference)
import jax
import jax.numpy as jnp
from jax import lax
from jax.experimental import pallas as pl
from jax.experimental.pallas import tpu as pltpu

M = 3072
N = 3072
G = 6144
CH = 256
T = M // CH


def kernel(x):
    def body(x_hbm, out_hbm, xt, out_t, row_halo, col_halo,
             load_sems, store_sems, row_sems, col_send_sems, col_recv_sems):
        my_x = lax.axis_index("x")
        my_y = lax.axis_index("y")

        def start_load(k):
            slot = k % 2
            if k == 0:
                src = x_hbm.at[pl.ds(0, CH + 8), :]
                dst = xt.at[slot, pl.ds(0, CH + 8), :]
            elif k == T - 1:
                src = x_hbm.at[pl.ds(k * CH - 8, CH + 8), :]
                dst = xt.at[slot, pl.ds(0, CH + 8), :]
            else:
                src = x_hbm.at[pl.ds(k * CH - 8, CH + 16), :]
                dst = xt.at[slot, pl.ds(0, CH + 16), :]
            cp = pltpu.make_async_copy(src, dst, load_sems.at[slot])
            cp.start()
            return cp

        loads = {0: start_load(0)}

        barrier = pltpu.get_barrier_semaphore()
        for nbr in ((1 - my_x, my_y), (my_x, 1 - my_y)):
            pl.semaphore_signal(barrier, inc=1, device_id=nbr,
                                device_id_type=pl.DeviceIdType.MESH)
        pl.semaphore_wait(barrier, 2)

        row_off = pl.multiple_of(jnp.where(my_x == 0, M - 8, 0), 8)
        row_rdma = pltpu.make_async_remote_copy(
            src_ref=x_hbm.at[pl.ds(row_off, 8), :],
            dst_ref=row_halo,
            send_sem=row_sems.at[0],
            recv_sem=row_sems.at[1],
            device_id=(1 - my_x, my_y),
            device_id_type=pl.DeviceIdType.MESH)
        row_rdma.start()

        col_off = pl.multiple_of(jnp.where(my_y == 0, N - 128, 0), 128)
        col_rdmas = []
        for k in range(T):
            r0 = k * CH
            cp = pltpu.make_async_remote_copy(
                src_ref=x_hbm.at[pl.ds(r0, CH), pl.ds(col_off, 128)],
                dst_ref=col_halo.at[pl.ds(r0, CH), :],
                send_sem=col_send_sems.at[k],
                recv_sem=col_recv_sems.at[k],
                device_id=(my_x, 1 - my_y),
                device_id_type=pl.DeviceIdType.MESH)
            cp.start()
            col_rdmas.append(cp)

        row_rdma.wait()

        stores = {}
        for k in range(T):
            slot = k % 2
            if k + 1 < T:
                loads[k + 1] = start_load(k + 1)
            loads[k].wait()
            col_rdmas[k].wait_recv()
            if k >= 2:
                stores[k - 2].wait()

            r0 = k * CH
            base = 0 if k == 0 else 8
            xtv = xt[slot, :, :]
            c = lax.slice(xtv, (base, 0), (base + CH, N))
            if k == 0:
                n = jnp.concatenate(
                    [row_halo[7:8, :], lax.slice(xtv, (0, 0), (CH - 1, N))],
                    axis=0)
            else:
                n = lax.slice(xtv, (base - 1, 0), (base - 1 + CH, N))
            if k == T - 1:
                s = jnp.concatenate(
                    [lax.slice(xtv, (base + 1, 0), (base + CH, N)),
                     row_halo[0:1, :]],
                    axis=0)
            else:
                s = lax.slice(xtv, (base + 1, 0), (base + 1 + CH, N))
            chv = col_halo[r0:r0 + CH, :]
            w = jnp.concatenate(
                [lax.slice(chv, (0, 127), (CH, 128)),
                 lax.slice(c, (0, 0), (CH, N - 1))],
                axis=1)
            e = jnp.concatenate(
                [lax.slice(c, (0, 1), (CH, N)),
                 lax.slice(chv, (0, 0), (CH, 1))],
                axis=1)
            stencil = 0.5 * c + 0.125 * (n + s + w + e)
            gi = lax.broadcasted_iota(jnp.int32, (CH, N), 0) + my_x * M + r0
            gj = lax.broadcasted_iota(jnp.int32, (CH, N), 1) + my_y * N
            interior = (gi > 0) & (gi < G - 1) & (gj > 0) & (gj < G - 1)
            out_t[slot, :, :] = jnp.where(interior, stencil, c)

            st = pltpu.make_async_copy(
                out_t.at[slot],
                out_hbm.at[pl.ds(r0, CH), :],
                store_sems.at[slot])
            st.start()
            stores[k] = st

        for k in range(T):
            col_rdmas[k].wait_send()
        stores[T - 2].wait()
        stores[T - 1].wait()

    return pl.pallas_call(
        body,
        out_shape=jax.ShapeDtypeStruct((M, N), jnp.float32),
        in_specs=[pl.BlockSpec(memory_space=pl.ANY)],
        out_specs=pl.BlockSpec(memory_space=pl.ANY),
        scratch_shapes=[
            pltpu.VMEM((2, CH + 16, N), jnp.float32),
            pltpu.VMEM((2, CH, N), jnp.float32),
            pltpu.VMEM((8, N), jnp.float32),
            pltpu.VMEM((M, 128), jnp.float32),
            pltpu.SemaphoreType.DMA((2,)),
            pltpu.SemaphoreType.DMA((2,)),
            pltpu.SemaphoreType.DMA((2,)),
            pltpu.SemaphoreType.DMA((T,)),
            pltpu.SemaphoreType.DMA((T,)),
        ],
        compiler_params=pltpu.CompilerParams(collective_id=0),
    )(x)


# device time: 68283 ns/iter; 1.0920x vs baseline; 1.0920x over previous
import jax
import jax.numpy as jnp
from jax import lax
from jax.experimental import pallas as pl
from jax.experimental.pallas import tpu as pltpu

M = 3072
N = 3072
G = 6144
CH = 256
T = M // CH


def kernel(x):
    def body(x_hbm, out_hbm, xt, out_t, row_halo, col_halo,
             load_sems, store_sems, row_sems, col_send_sems, col_recv_sems):
        my_x = lax.axis_index("x")
        my_y = lax.axis_index("y")

        def start_load(k):
            slot = k % 2
            if k == 0:
                src = x_hbm.at[pl.ds(0, CH + 8), :]
                dst = xt.at[slot, pl.ds(0, CH + 8), :]
            elif k == T - 1:
                src = x_hbm.at[pl.ds(k * CH - 8, CH + 8), :]
                dst = xt.at[slot, pl.ds(0, CH + 8), :]
            else:
                src = x_hbm.at[pl.ds(k * CH - 8, CH + 16), :]
                dst = xt.at[slot, pl.ds(0, CH + 16), :]
            cp = pltpu.make_async_copy(src, dst, load_sems.at[slot])
            cp.start()
            return cp

        loads = {0: start_load(0)}

        barrier = pltpu.get_barrier_semaphore()
        for nbr in ((1 - my_x, my_y), (my_x, 1 - my_y)):
            pl.semaphore_signal(barrier, inc=1, device_id=nbr,
                                device_id_type=pl.DeviceIdType.MESH)
        pl.semaphore_wait(barrier, 2)

        row_off = pl.multiple_of(jnp.where(my_x == 0, M - 8, 0), 8)
        row_rdma = pltpu.make_async_remote_copy(
            src_ref=x_hbm.at[pl.ds(row_off, 8), :],
            dst_ref=row_halo,
            send_sem=row_sems.at[0],
            recv_sem=row_sems.at[1],
            device_id=(1 - my_x, my_y),
            device_id_type=pl.DeviceIdType.MESH)
        row_rdma.start()

        col_off = pl.multiple_of(jnp.where(my_y == 0, N - 128, 0), 128)
        col_rdmas = []
        for k in range(T):
            r0 = k * CH
            cp = pltpu.make_async_remote_copy(
                src_ref=x_hbm.at[pl.ds(r0, CH), pl.ds(col_off, 128)],
                dst_ref=col_halo.at[pl.ds(r0, CH), :],
                send_sem=col_send_sems.at[k],
                recv_sem=col_recv_sems.at[k],
                device_id=(my_x, 1 - my_y),
                device_id_type=pl.DeviceIdType.MESH)
            cp.start()
            col_rdmas.append(cp)

        row_rdma.wait_recv()

        stores = {}
        for k in range(T):
            slot = k % 2
            if k + 1 < T:
                loads[k + 1] = start_load(k + 1)
            loads[k].wait()
            col_rdmas[k].wait_recv()
            if k >= 2:
                stores[k - 2].wait()

            r0 = k * CH
            base = 0 if k == 0 else 8
            c = xt[slot, base:base + CH, :]
            if k == 0:
                n = jnp.concatenate(
                    [row_halo[7:8, :], xt[slot, 0:CH - 1, :]], axis=0)
            else:
                n = xt[slot, base - 1:base - 1 + CH, :]
            if k == T - 1:
                s = jnp.concatenate(
                    [xt[slot, base + 1:base + CH, :], row_halo[0:1, :]],
                    axis=0)
            else:
                s = xt[slot, base + 1:base + 1 + CH, :]
            w = jnp.concatenate(
                [col_halo[r0:r0 + CH, 127:128],
                 xt[slot, base:base + CH, 0:N - 1]],
                axis=1)
            e = jnp.concatenate(
                [xt[slot, base:base + CH, 1:N],
                 col_halo[r0:r0 + CH, 0:1]],
                axis=1)
            out_t[slot, :, :] = 0.5 * c + 0.125 * (n + s + w + e)

            @pl.when(my_y == 0)
            def _():
                out_t[slot, :, 0:1] = xt[slot, base:base + CH, 0:1]

            @pl.when(my_y == 1)
            def _():
                out_t[slot, :, N - 1:N] = xt[slot, base:base + CH, N - 1:N]

            if k == 0:
                @pl.when(my_x == 0)
                def _():
                    out_t[slot, 0:1, :] = xt[slot, 0:1, :]
            if k == T - 1:
                @pl.when(my_x == 1)
                def _():
                    out_t[slot, CH - 1:CH, :] = xt[slot, base + CH - 1:base + CH, :]

            st = pltpu.make_async_copy(
                out_t.at[slot],
                out_hbm.at[pl.ds(r0, CH), :],
                store_sems.at[slot])
            st.start()
            stores[k] = st

        row_rdma.wait_send()
        for k in range(T):
            col_rdmas[k].wait_send()
        stores[T - 2].wait()
        stores[T - 1].wait()

    return pl.pallas_call(
        body,
        out_shape=jax.ShapeDtypeStruct((M, N), jnp.float32),
        in_specs=[pl.BlockSpec(memory_space=pl.ANY)],
        out_specs=pl.BlockSpec(memory_space=pl.ANY),
        scratch_shapes=[
            pltpu.VMEM((2, CH + 16, N), jnp.float32),
            pltpu.VMEM((2, CH, N), jnp.float32),
            pltpu.VMEM((8, N), jnp.float32),
            pltpu.VMEM((M, 128), jnp.float32),
            pltpu.SemaphoreType.DMA((2,)),
            pltpu.SemaphoreType.DMA((2,)),
            pltpu.SemaphoreType.DMA((2,)),
            pltpu.SemaphoreType.DMA((T,)),
            pltpu.SemaphoreType.DMA((T,)),
        ],
        compiler_params=pltpu.CompilerParams(collective_id=0),
    )(x)


# device time: 68132 ns/iter; 1.0944x vs baseline; 1.0022x over previous
import jax
import jax.numpy as jnp
from jax import lax
from jax.experimental import pallas as pl
from jax.experimental.pallas import tpu as pltpu

M = 3072
N = 3072
G = 6144
CH = 512
T = M // CH


def kernel(x):
    def body(x_hbm, out_hbm, xt, out_t, row_halo, col_halo,
             wsave, esave, fix_w, fix_e,
             load_sems, store_sems, row_sems, col_sems, fix_sems):
        my_x = lax.axis_index("x")
        my_y = lax.axis_index("y")

        def start_load(k):
            slot = k % 2
            if k == 0:
                src = x_hbm.at[pl.ds(0, CH + 8), :]
                dst = xt.at[slot, pl.ds(0, CH + 8), :]
            elif k == T - 1:
                src = x_hbm.at[pl.ds(k * CH - 8, CH + 8), :]
                dst = xt.at[slot, pl.ds(0, CH + 8), :]
            else:
                src = x_hbm.at[pl.ds(k * CH - 8, CH + 16), :]
                dst = xt.at[slot, pl.ds(0, CH + 16), :]
            cp = pltpu.make_async_copy(src, dst, load_sems.at[slot])
            cp.start()
            return cp

        loads = {0: start_load(0)}

        barrier = pltpu.get_barrier_semaphore()
        for nbr in ((1 - my_x, my_y), (my_x, 1 - my_y)):
            pl.semaphore_signal(barrier, inc=1, device_id=nbr,
                                device_id_type=pl.DeviceIdType.MESH)
        pl.semaphore_wait(barrier, 2)

        row_off = pl.multiple_of(jnp.where(my_x == 0, M - 8, 0), 8)
        row_rdma = pltpu.make_async_remote_copy(
            src_ref=x_hbm.at[pl.ds(row_off, 8), :],
            dst_ref=row_halo,
            send_sem=row_sems.at[0],
            recv_sem=row_sems.at[1],
            device_id=(1 - my_x, my_y),
            device_id_type=pl.DeviceIdType.MESH)
        row_rdma.start()

        col_off = pl.multiple_of(jnp.where(my_y == 0, N - 128, 0), 128)
        col_rdma = pltpu.make_async_remote_copy(
            src_ref=x_hbm.at[:, pl.ds(col_off, 128)],
            dst_ref=col_halo,
            send_sem=col_sems.at[0],
            recv_sem=col_sems.at[1],
            device_id=(my_x, 1 - my_y),
            device_id_type=pl.DeviceIdType.MESH)
        col_rdma.start()

        row_rdma.wait_recv()

        stores = {}
        for k in range(T):
            slot = k % 2
            if k + 1 < T:
                loads[k + 1] = start_load(k + 1)
            loads[k].wait()
            if k >= 2:
                stores[k - 2].wait()

            r0 = k * CH
            base = 0 if k == 0 else 8
            c = xt[slot, base:base + CH, :]
            if k == 0:
                n = jnp.concatenate(
                    [row_halo[7:8, :], xt[slot, 0:CH - 1, :]], axis=0)
            else:
                n = xt[slot, base - 1:base - 1 + CH, :]
            if k == T - 1:
                s = jnp.concatenate(
                    [xt[slot, base + 1:base + CH, :], row_halo[0:1, :]],
                    axis=0)
            else:
                s = xt[slot, base + 1:base + 1 + CH, :]
            w = jnp.concatenate([c[:, 0:1], c[:, 0:N - 1]], axis=1)
            e = jnp.concatenate([c[:, 1:N], c[:, N - 1:N]], axis=1)
            out_t[slot, :, :] = 0.5 * c + 0.125 * (n + s + w + e)

            wsave[r0:r0 + CH, :] = xt[slot, base:base + CH, 0:256]
            esave[r0:r0 + CH, :] = xt[slot, base:base + CH, N - 256:N]

            if k == 0:
                @pl.when(my_x == 0)
                def _():
                    out_t[slot, 0:1, :] = xt[slot, 0:1, :]
            if k == T - 1:
                @pl.when(my_x == 1)
                def _():
                    out_t[slot, CH - 1:CH, :] = \
                        xt[slot, base + CH - 1:base + CH, :]

            st = pltpu.make_async_copy(
                out_t.at[slot],
                out_hbm.at[pl.ds(r0, CH), :],
                store_sems.at[slot])
            st.start()
            stores[k] = st

        col_rdma.wait_recv()

        cw = wsave[:, 0:128]
        wv = jnp.concatenate([col_halo[:, 127:128], wsave[:, 0:127]], axis=1)
        ev = wsave[:, 1:129]
        nv = jnp.concatenate([row_halo[7:8, 0:128], wsave[0:M - 1, 0:128]],
                             axis=0)
        sv = jnp.concatenate([wsave[1:M, 0:128], row_halo[0:1, 0:128]],
                             axis=0)
        fix_w[:, :] = 0.5 * cw + 0.125 * (nv + sv + wv + ev)

        @pl.when(my_y == 0)
        def _():
            fix_w[:, 0:1] = wsave[:, 0:1]

        @pl.when(my_x == 0)
        def _():
            fix_w[0:1, :] = wsave[0:1, 0:128]

        @pl.when(my_x == 1)
        def _():
            fix_w[M - 1:M, :] = wsave[M - 1:M, 0:128]

        ce = esave[:, 128:256]
        wv2 = esave[:, 127:255]
        ev2 = jnp.concatenate([esave[:, 129:256], col_halo[:, 0:1]], axis=1)
        nv2 = jnp.concatenate(
            [row_halo[7:8, N - 128:N], esave[0:M - 1, 128:256]], axis=0)
        sv2 = jnp.concatenate(
            [esave[1:M, 128:256], row_halo[0:1, N - 128:N]], axis=0)
        fix_e[:, :] = 0.5 * ce + 0.125 * (nv2 + sv2 + wv2 + ev2)

        @pl.when(my_y == 1)
        def _():
            fix_e[:, 127:128] = esave[:, 255:256]

        @pl.when(my_x == 0)
        def _():
            fix_e[0:1, :] = esave[0:1, 128:256]

        @pl.when(my_x == 1)
        def _():
            fix_e[M - 1:M, :] = esave[M - 1:M, 128:256]

        stores[T - 2].wait()
        stores[T - 1].wait()
        fw = pltpu.make_async_copy(fix_w, out_hbm.at[:, pl.ds(0, 128)],
                                   fix_sems.at[0])
        fw.start()
        fe = pltpu.make_async_copy(fix_e, out_hbm.at[:, pl.ds(N - 128, 128)],
                                   fix_sems.at[1])
        fe.start()
        fw.wait()
        fe.wait()
        row_rdma.wait_send()
        col_rdma.wait_send()

    return pl.pallas_call(
        body,
        out_shape=jax.ShapeDtypeStruct((M, N), jnp.float32),
        in_specs=[pl.BlockSpec(memory_space=pl.ANY)],
        out_specs=pl.BlockSpec(memory_space=pl.ANY),
        scratch_shapes=[
            pltpu.VMEM((2, CH + 16, N), jnp.float32),
            pltpu.VMEM((2, CH, N), jnp.float32),
            pltpu.VMEM((8, N), jnp.float32),
            pltpu.VMEM((M, 128), jnp.float32),
            pltpu.VMEM((M, 256), jnp.float32),
            pltpu.VMEM((M, 256), jnp.float32),
            pltpu.VMEM((M, 128), jnp.float32),
            pltpu.VMEM((M, 128), jnp.float32),
            pltpu.SemaphoreType.DMA((2,)),
            pltpu.SemaphoreType.DMA((2,)),
            pltpu.SemaphoreType.DMA((2,)),
            pltpu.SemaphoreType.DMA((2,)),
            pltpu.SemaphoreType.DMA((2,)),
        ],
        compiler_params=pltpu.CompilerParams(collective_id=0,
                                     vmem_limit_bytes=100 * 1024 * 1024),
    )(x)
